# Initial kernel scaffold; baseline (speedup 1.0000x reference)
import jax
import jax.numpy as jnp
from jax.experimental import pallas as pl


def kernel(y, snr, gamma):
    raise NotImplementedError("write your pallas kernel here")


# speedup vs baseline: 34.1812x; 34.1812x over previous
import functools
import math

import jax
import jax.numpy as jnp
import numpy as np
from jax import lax
from jax.experimental import pallas as pl
from jax.experimental.pallas import tpu as pltpu
from jax.experimental.pallas import tpu_sc as plsc

N_STATES = 16
MEM_LEN = 4
T_LEN = 2048
BATCH = 512
LANES = 16
N_WORKERS = 32
CHUNK = 64
N_CHUNKS = T_LEN // CHUNK


def _treesum(xs):
    xs = list(xs)
    while len(xs) > 1:
        nxt = [xs[i] + xs[i + 1] for i in range(0, len(xs) - 1, 2)]
        if len(xs) % 2:
            nxt.append(xs[-1])
        xs = nxt
    return xs[0]


def _row(ref, r):
    return ref[pl.ds(r * LANES, LANES)]


def _bcjr_sc_kernel(y_hbm, par_hbm, out_hbm, y_v, dec_v, ckpt_v, abuf_v,
                    ebuf_v, par_v):
    S = N_STATES
    wid = lax.axis_index("s") * 2 + lax.axis_index("c")
    pltpu.sync_copy(par_hbm, par_v)
    pltpu.sync_copy(y_hbm.at[wid], y_v)

    spv = [_row(par_v, s) for s in range(S)]
    ninv2 = _row(par_v, S)
    nlogc = _row(par_v, S + 1)

    def branch_metrics(yv):
        es = []
        for s in range(S):
            d = yv - spv[s]
            es.append(jnp.exp(d * d * ninv2 + nlogc))
        return es

    def alpha_advance(a, es):
        pair = [a[j] + a[j + 8] for j in range(8)]
        na = [pair[s // 2] * es[s] for s in range(S)]
        tot = _treesum(na)
        return tuple(x / tot for x in na)

    one = jnp.full((LANES,), 1.0, jnp.float32)
    zero = jnp.full((LANES,), 0.0, jnp.float32)
    onehot0 = (one,) + (zero,) * (S - 1)

    def fwd_chunk(c, a):
        for s in range(S):
            ckpt_v[pl.ds((c * S + s) * LANES, LANES)] = a[s]

        def step(i, a):
            es = branch_metrics(_row(y_v, c * CHUNK + i))
            return alpha_advance(a, es)

        return lax.fori_loop(0, CHUNK, step, a)

    lax.fori_loop(0, N_CHUNKS, fwd_chunk, onehot0)

    def bwd_chunk(ci, b):
        c = N_CHUNKS - 1 - ci
        a0 = tuple(_row(ckpt_v, c * S + s) for s in range(S))

        def re_step(i, a):
            es = branch_metrics(_row(y_v, c * CHUNK + i))
            for s in range(S):
                abuf_v[pl.ds((i * S + s) * LANES, LANES)] = a[s]
                ebuf_v[pl.ds((i * S + s) * LANES, LANES)] = es[s]
            return alpha_advance(a, es)

        lax.fori_loop(0, CHUNK, re_step, a0)

        def b_step(i, b):
            j = CHUNK - 1 - i
            es = [_row(ebuf_v, j * S + s) for s in range(S)]
            av = [_row(abuf_v, j * S + s) for s in range(S)]
            q = [b[2 * k] + b[2 * k + 1] for k in range(8)]
            nb = [q[s % 8] * es[s] for s in range(S)]
            tot = _treesum(nb)
            nb = [x / tot for x in nb]
            g = [es[k] * nb[k] for k in range(S)]
            up = _treesum([av[s] * g[(2 * s) % S] for s in range(S)])
            down = _treesum([av[s] * g[(2 * s + 1) % S] for s in range(S)])
            dec_v[pl.ds((c * CHUNK + j) * LANES, LANES)] = jnp.where(
                up < down, one, zero)
            return tuple(nb)

        return lax.fori_loop(0, CHUNK, b_step, b)

    lax.fori_loop(0, N_CHUNKS, bwd_chunk, onehot0)

    pltpu.sync_copy(dec_v, out_hbm.at[wid])


@jax.jit
def _bcjr_call(yt, params):
    mesh = plsc.VectorSubcoreMesh(core_axis_name="c", subcore_axis_name="s")
    f = functools.partial(
        pl.kernel,
        mesh=mesh,
        out_type=jax.ShapeDtypeStruct((N_WORKERS, T_LEN * LANES), jnp.float32),
        scratch_types=[
            pltpu.VMEM((T_LEN * LANES,), jnp.float32),
            pltpu.VMEM((T_LEN * LANES,), jnp.float32),
            pltpu.VMEM((N_CHUNKS * N_STATES * LANES,), jnp.float32),
            pltpu.VMEM((CHUNK * N_STATES * LANES,), jnp.float32),
            pltpu.VMEM((CHUNK * N_STATES * LANES,), jnp.float32),
            pltpu.VMEM(((N_STATES + 2) * LANES,), jnp.float32),
        ],
    )(_bcjr_sc_kernel)
    return f(yt, params)


def kernel(y, snr, gamma):
    B, T = y.shape
    m = MEM_LEN
    gamma_f = jnp.asarray(gamma, jnp.float32)
    h = jnp.exp(-gamma_f * jnp.arange(m, dtype=jnp.float32))
    bits = np.unpackbits(np.arange(N_STATES, dtype=np.uint8).reshape(-1, 1),
                         axis=1)[:, -m:].astype(np.float32)
    symbols = jnp.asarray(1.0 - 2.0 * bits)
    sp = symbols @ h[::-1]
    sigma = 10.0 ** (-jnp.asarray(snr, jnp.float32) / 10.0)
    ninv2 = -1.0 / (2.0 * sigma * sigma)
    nlogc = -jnp.log(jnp.asarray(math.sqrt(2.0 * math.pi), jnp.float32) * sigma)
    params = jnp.concatenate(
        [jnp.broadcast_to(sp[:, None], (N_STATES, LANES)),
         jnp.full((1, LANES), ninv2, jnp.float32),
         jnp.full((1, LANES), nlogc, jnp.float32)], axis=0)

    yt = y.reshape(N_WORKERS, LANES, T).transpose(0, 2, 1)
    raw = _bcjr_call(yt.reshape(N_WORKERS, T * LANES),
                     params.reshape((N_STATES + 2) * LANES))
    raw = raw.reshape(N_WORKERS, T, LANES).transpose(0, 2, 1).reshape(B, T)
    out = jnp.concatenate(
        [jnp.zeros((B, m - 1), y.dtype), raw[:, : T - (m - 1)]], axis=1)
    return out


# speedup vs baseline: 36.6538x; 1.0723x over previous
import functools
import math

import jax
import jax.numpy as jnp
import numpy as np
from jax import lax
from jax.experimental import pallas as pl
from jax.experimental.pallas import tpu as pltpu
from jax.experimental.pallas import tpu_sc as plsc

N_STATES = 16
MEM_LEN = 4
T_LEN = 2048
BATCH = 512
LANES = 16
N_WORKERS = 32
CHUNK = 64
N_CHUNKS = T_LEN // CHUNK


def _treesum(xs):
    xs = list(xs)
    while len(xs) > 1:
        nxt = [xs[i] + xs[i + 1] for i in range(0, len(xs) - 1, 2)]
        if len(xs) % 2:
            nxt.append(xs[-1])
        xs = nxt
    return xs[0]


def _row(ref, r):
    return ref[pl.ds(r * LANES, LANES)]


def _bcjr_sc_kernel(y_hbm, par_hbm, out_hbm, y_v, dec_v, ckpt_v, abuf_v,
                    ebuf_v, par_v):
    S = N_STATES
    wid = lax.axis_index("s") * 2 + lax.axis_index("c")
    pltpu.sync_copy(par_hbm, par_v)
    pltpu.sync_copy(y_hbm.at[wid], y_v)

    c1v = [_row(par_v, s) for s in range(S)]
    c2v = [_row(par_v, S + s) for s in range(S)]
    ninv2 = _row(par_v, 2 * S)
    nlogc = _row(par_v, 2 * S + 1)

    def branch_metrics(yv):
        u = yv * yv * ninv2 + nlogc
        return [jnp.exp(c1v[s] * yv + (c2v[s] + u)) for s in range(S)]

    def alpha_products(a, es):
        pair = [a[j] + a[j + 8] for j in range(8)]
        return [pair[s // 2] * es[s] for s in range(S)]

    def alpha_advance(a, es):
        na = alpha_products(a, es)
        r = 1.0 / _treesum(na)
        return tuple(x * r for x in na)

    one = jnp.full((LANES,), 1.0, jnp.float32)
    zero = jnp.full((LANES,), 0.0, jnp.float32)
    onehot0 = (one,) + (zero,) * (S - 1)

    def fwd_chunk(c, a):
        for s in range(S):
            ckpt_v[pl.ds((c * S + s) * LANES, LANES)] = a[s]

        def step(i, a):
            es = branch_metrics(_row(y_v, c * CHUNK + i))
            return alpha_advance(a, es)

        return lax.fori_loop(0, CHUNK, step, a)

    lax.fori_loop(0, N_CHUNKS, fwd_chunk, onehot0)

    def bwd_chunk(ci, b):
        c = N_CHUNKS - 1 - ci
        a0 = tuple(_row(ckpt_v, c * S + s) for s in range(S))

        def re_step(i, a):
            es = branch_metrics(_row(y_v, c * CHUNK + i))
            na = alpha_products(a, es)
            for s in range(S):
                abuf_v[pl.ds((i * S + s) * LANES, LANES)] = na[s]
                ebuf_v[pl.ds((i * S + s) * LANES, LANES)] = es[s]
            r = 1.0 / _treesum(na)
            return tuple(x * r for x in na)

        lax.fori_loop(0, CHUNK, re_step, a0)

        def b_step(i, b):
            j = CHUNK - 1 - i
            es = [_row(ebuf_v, j * S + s) for s in range(S)]
            nav = [_row(abuf_v, j * S + s) for s in range(S)]
            q = [b[2 * k] + b[2 * k + 1] for k in range(8)]
            nb = [q[s % 8] * es[s] for s in range(S)]
            r = 1.0 / _treesum(nb)
            nb = [x * r for x in nb]
            up = _treesum([nav[2 * j2] * nb[2 * j2] for j2 in range(8)])
            down = _treesum([nav[2 * j2 + 1] * nb[2 * j2 + 1] for j2 in range(8)])
            dec_v[pl.ds((c * CHUNK + j) * LANES, LANES)] = jnp.where(
                up < down, one, zero)
            return tuple(nb)

        return lax.fori_loop(0, CHUNK, b_step, b)

    lax.fori_loop(0, N_CHUNKS, bwd_chunk, onehot0)

    pltpu.sync_copy(dec_v, out_hbm.at[wid])


@jax.jit
def _bcjr_call(yt, params):
    mesh = plsc.VectorSubcoreMesh(core_axis_name="c", subcore_axis_name="s")
    f = functools.partial(
        pl.kernel,
        mesh=mesh,
        out_type=jax.ShapeDtypeStruct((N_WORKERS, T_LEN * LANES), jnp.float32),
        scratch_types=[
            pltpu.VMEM((T_LEN * LANES,), jnp.float32),
            pltpu.VMEM((T_LEN * LANES,), jnp.float32),
            pltpu.VMEM((N_CHUNKS * N_STATES * LANES,), jnp.float32),
            pltpu.VMEM((CHUNK * N_STATES * LANES,), jnp.float32),
            pltpu.VMEM((CHUNK * N_STATES * LANES,), jnp.float32),
            pltpu.VMEM(((2 * N_STATES + 2) * LANES,), jnp.float32),
        ],
    )(_bcjr_sc_kernel)
    return f(yt, params)


def kernel(y, snr, gamma):
    B, T = y.shape
    m = MEM_LEN
    gamma_f = jnp.asarray(gamma, jnp.float32)
    h = jnp.exp(-gamma_f * jnp.arange(m, dtype=jnp.float32))
    bits = np.unpackbits(np.arange(N_STATES, dtype=np.uint8).reshape(-1, 1),
                         axis=1)[:, -m:].astype(np.float32)
    symbols = jnp.asarray(1.0 - 2.0 * bits)
    sp = symbols @ h[::-1]
    sigma = 10.0 ** (-jnp.asarray(snr, jnp.float32) / 10.0)
    ninv2 = -1.0 / (2.0 * sigma * sigma)
    nlogc = -jnp.log(jnp.asarray(math.sqrt(2.0 * math.pi), jnp.float32) * sigma)
    c1 = -2.0 * ninv2 * sp
    c2 = ninv2 * sp * sp
    params = jnp.concatenate(
        [jnp.broadcast_to(c1[:, None], (N_STATES, LANES)),
         jnp.broadcast_to(c2[:, None], (N_STATES, LANES)),
         jnp.full((1, LANES), ninv2, jnp.float32),
         jnp.full((1, LANES), nlogc, jnp.float32)], axis=0)

    yt = y.reshape(N_WORKERS, LANES, T).transpose(0, 2, 1)
    raw = _bcjr_call(yt.reshape(N_WORKERS, T * LANES),
                     params.reshape((2 * N_STATES + 2) * LANES))
    raw = raw.reshape(N_WORKERS, T, LANES).transpose(0, 2, 1).reshape(B, T)
    out = jnp.concatenate(
        [jnp.zeros((B, m - 1), y.dtype), raw[:, : T - (m - 1)]], axis=1)
    return out


# speedup vs baseline: 45.3131x; 1.2362x over previous
import functools
import math

import jax
import jax.numpy as jnp
import numpy as np
from jax import lax
from jax.experimental import pallas as pl
from jax.experimental.pallas import tpu as pltpu
from jax.experimental.pallas import tpu_sc as plsc

N_STATES = 16
MEM_LEN = 4
T_LEN = 2048
BATCH = 512
LANES = 16
N_WORKERS = 32
CHUNK = 64
N_CHUNKS = T_LEN // CHUNK


def _treesum(xs):
    xs = list(xs)
    while len(xs) > 1:
        nxt = [xs[i] + xs[i + 1] for i in range(0, len(xs) - 1, 2)]
        if len(xs) % 2:
            nxt.append(xs[-1])
        xs = nxt
    return xs[0]


def _row(ref, r):
    return ref[pl.ds(r * LANES, LANES)]


def _bcjr_sc_kernel(y_hbm, par_hbm, out_hbm, y_v, dec_v, ckpt_v, abuf_v,
                    ebuf_v, par_v):
    S = N_STATES
    wid = lax.axis_index("s") * 2 + lax.axis_index("c")
    pltpu.sync_copy(par_hbm, par_v)
    pltpu.sync_copy(y_hbm.at[wid], y_v)

    c1v = [_row(par_v, s) for s in range(S)]
    c2v = [_row(par_v, S + s) for s in range(S)]
    ninv2 = _row(par_v, 2 * S)
    nlogc = _row(par_v, 2 * S + 1)

    def yload(t):
        return _row(y_v, t)

    def branch_metrics(yv):
        u = yv * yv * ninv2 + nlogc
        return [jnp.exp2(c1v[s] * yv + (c2v[s] + u)) for s in range(S)]

    def alpha_products(a, es):
        pair = [a[j] + a[j + 8] for j in range(8)]
        return [pair[s // 2] * es[s] for s in range(S)]

    def normalize(na):
        tot = _treesum(na)
        bits = lax.bitcast_convert_type(tot, jnp.uint32)
        rb = jnp.uint32(254 << 23) - (bits & jnp.uint32(0xFF800000))
        r = lax.bitcast_convert_type(rb, jnp.float32)
        return tuple(x * r for x in na)

    one = jnp.full((LANES,), 1.0, jnp.float32)
    zero = jnp.full((LANES,), 0.0, jnp.float32)
    onehot0 = (one,) + (zero,) * (S - 1)

    for k in range(MEM_LEN - 1):
        dec_v[pl.ds(k * LANES, LANES)] = zero

    def fwd_chunk(c, a):
        for s in range(S):
            ckpt_v[pl.ds((c * S + s) * LANES, LANES)] = a[s]

        def step8(i, a):
            t0 = c * CHUNK + 8 * i
            na = a
            for k in range(8):
                na = alpha_products(na, branch_metrics(yload(t0 + k)))
            return normalize(na)

        return lax.fori_loop(0, CHUNK // 8, step8, a)

    lax.fori_loop(0, N_CHUNKS, fwd_chunk, onehot0)

    def bwd_chunk(ci, b):
        c = N_CHUNKS - 1 - ci
        a0 = tuple(_row(ckpt_v, c * S + s) for s in range(S))

        def re_step8(i, a):
            na = a
            for k in range(8):
                es = branch_metrics(yload(c * CHUNK + 8 * i + k))
                na = alpha_products(na, es)
                for s in range(S):
                    abuf_v[pl.ds(((8 * i + k) * S + s) * LANES, LANES)] = na[s]
                    ebuf_v[pl.ds(((8 * i + k) * S + s) * LANES, LANES)] = es[s]
            return normalize(na)

        lax.fori_loop(0, CHUNK // 8, re_step8, a0)

        def decode(nav, nb, t):
            up = _treesum([nav[2 * k] * nb[2 * k] for k in range(8)])
            down = _treesum([nav[2 * k + 1] * nb[2 * k + 1] for k in range(8)])
            dec_v[pl.ds((t + MEM_LEN - 1) * LANES, LANES)] = jnp.where(
                up < down, one, zero)

        def beta_products(b, es):
            q = [b[2 * k] + b[2 * k + 1] for k in range(8)]
            return [q[s % 8] * es[s] for s in range(S)]

        def b_step8(i, b):
            nb = tuple(b)
            for k in range(7):
                j = CHUNK - 1 - 8 * i - k
                es = [_row(ebuf_v, j * S + s) for s in range(S)]
                nav = [_row(abuf_v, j * S + s) for s in range(S)]
                nb = beta_products(nb, es)
                decode(nav, nb, c * CHUNK + j)
            j = CHUNK - 8 - 8 * i
            es = [_row(ebuf_v, j * S + s) for s in range(S)]
            nav = [_row(abuf_v, j * S + s) for s in range(S)]
            nb = normalize(beta_products(nb, es))
            decode(nav, nb, c * CHUNK + j)
            return tuple(nb)

        return lax.fori_loop(0, CHUNK // 8, b_step8, tuple(b))

    lax.fori_loop(0, N_CHUNKS, bwd_chunk, onehot0)

    pltpu.sync_copy(dec_v.at[pl.ds(0, T_LEN * LANES)], out_hbm.at[wid])


@jax.jit
def _bcjr_call(yt, params):
    mesh = plsc.VectorSubcoreMesh(core_axis_name="c", subcore_axis_name="s")
    f = functools.partial(
        pl.kernel,
        mesh=mesh,
        out_type=jax.ShapeDtypeStruct((N_WORKERS, T_LEN * LANES), jnp.float32),
        scratch_types=[
            pltpu.VMEM((T_LEN * LANES,), jnp.float32),
            pltpu.VMEM(((T_LEN + MEM_LEN - 1) * LANES,), jnp.float32),
            pltpu.VMEM((N_CHUNKS * N_STATES * LANES,), jnp.float32),
            pltpu.VMEM((CHUNK * N_STATES * LANES,), jnp.float32),
            pltpu.VMEM((CHUNK * N_STATES * LANES,), jnp.float32),
            pltpu.VMEM(((2 * N_STATES + 2) * LANES,), jnp.float32),
        ],
    )(_bcjr_sc_kernel)
    return f(yt, params)


def kernel(y, snr, gamma):
    m = MEM_LEN
    gamma_f = jnp.asarray(gamma, jnp.float32)
    h = jnp.exp(-gamma_f * jnp.arange(m, dtype=jnp.float32))
    bits = np.unpackbits(np.arange(N_STATES, dtype=np.uint8).reshape(-1, 1),
                         axis=1)[:, -m:].astype(np.float32)
    symbols = jnp.asarray(1.0 - 2.0 * bits)
    sp = symbols @ h[::-1]
    sigma = 10.0 ** (-jnp.asarray(snr, jnp.float32) / 10.0)
    ninv2 = -1.0 / (2.0 * sigma * sigma)
    nlogc = -jnp.log(jnp.asarray(math.sqrt(2.0 * math.pi), jnp.float32) * sigma)
    log2e = jnp.float32(1.4426950408889634)
    c1 = -2.0 * ninv2 * sp * log2e
    c2 = ninv2 * sp * sp * log2e
    params = jnp.concatenate(
        [jnp.broadcast_to(c1[:, None], (N_STATES, LANES)),
         jnp.broadcast_to(c2[:, None], (N_STATES, LANES)),
         jnp.full((1, LANES), ninv2 * log2e, jnp.float32),
         jnp.full((1, LANES), nlogc * log2e, jnp.float32)], axis=0)

    B, T = y.shape
    yt = y.reshape(N_WORKERS, LANES, T).transpose(0, 2, 1)
    raw = _bcjr_call(yt.reshape(N_WORKERS, T * LANES),
                     params.reshape((2 * N_STATES + 2) * LANES))
    return raw.reshape(N_WORKERS, T, LANES).transpose(0, 2, 1).reshape(B, T)


# speedup vs baseline: 46.7546x; 1.0318x over previous
import functools
import math

import jax
import jax.numpy as jnp
import numpy as np
from jax import lax
from jax.experimental import pallas as pl
from jax.experimental.pallas import tpu as pltpu
from jax.experimental.pallas import tpu_sc as plsc

N_STATES = 16
MEM_LEN = 4
T_LEN = 2048
BATCH = 512
LANES = 16
N_WORKERS = 32
CHUNK = 64
N_CHUNKS = T_LEN // CHUNK


def _treesum(xs):
    xs = list(xs)
    while len(xs) > 1:
        nxt = [xs[i] + xs[i + 1] for i in range(0, len(xs) - 1, 2)]
        if len(xs) % 2:
            nxt.append(xs[-1])
        xs = nxt
    return xs[0]


def _row(ref, r):
    return ref[pl.ds(r * LANES, LANES)]


def _bcjr_sc_kernel(y_hbm, par_hbm, out_hbm, y_v, dec_v, ckpt_v, abuf_v,
                    ebuf_v, par_v):
    S = N_STATES
    wid = lax.axis_index("s") * 2 + lax.axis_index("c")
    pltpu.sync_copy(par_hbm, par_v)
    pltpu.sync_copy(y_hbm.at[wid], y_v)

    c1v = [_row(par_v, s) for s in range(S)]
    c2v = [_row(par_v, S + s) for s in range(S)]
    ninv2 = _row(par_v, 2 * S)
    nlogc = _row(par_v, 2 * S + 1)

    def yload(t):
        return _row(y_v, t)

    def branch_metrics(yv):
        u = yv * yv * ninv2 + nlogc
        return [jnp.exp(c1v[s] * yv + (c2v[s] + u)) for s in range(S)]

    def alpha_products(a, es):
        pair = [a[j] + a[j + 8] for j in range(8)]
        return [pair[s // 2] * es[s] for s in range(S)]

    def normalize(na):
        tot = _treesum(na)
        bits = lax.bitcast_convert_type(tot, jnp.uint32)
        rb = jnp.uint32(254 << 23) - (bits & jnp.uint32(0xFF800000))
        r = lax.bitcast_convert_type(rb, jnp.float32)
        return tuple(x * r for x in na)

    one = jnp.full((LANES,), 1.0, jnp.float32)
    zero = jnp.full((LANES,), 0.0, jnp.float32)
    onehot0 = (one,) + (zero,) * (S - 1)

    for k in range(MEM_LEN - 1):
        dec_v[pl.ds(k * LANES, LANES)] = zero

    def fwd_chunk(c, a):
        for s in range(S):
            ckpt_v[pl.ds((c * S + s) * LANES, LANES)] = a[s]

        def step8(i, a):
            t0 = c * CHUNK + 8 * i
            na = a
            for k in range(8):
                na = alpha_products(na, branch_metrics(yload(t0 + k)))
            return normalize(na)

        return lax.fori_loop(0, CHUNK // 8, step8, a)

    lax.fori_loop(0, N_CHUNKS, fwd_chunk, onehot0)

    def bwd_chunk(ci, b):
        c = N_CHUNKS - 1 - ci
        a0 = tuple(_row(ckpt_v, c * S + s) for s in range(S))

        def re_step8(i, a):
            na = a
            for k in range(8):
                es = branch_metrics(yload(c * CHUNK + 8 * i + k))
                pair = [na[j] + na[j + 8] for j in range(8)]
                na = [pair[s // 2] * es[s] for s in range(S)]
                for j in range(8):
                    abuf_v[pl.ds(((8 * i + k) * 8 + j) * LANES, LANES)] = pair[j]
                for s in range(S):
                    ebuf_v[pl.ds(((8 * i + k) * S + s) * LANES, LANES)] = es[s]
            return normalize(na)

        lax.fori_loop(0, CHUNK // 8, re_step8, a0)

        def decode(pr, es, nb, t):
            g = [es[k] * nb[k] for k in range(S)]
            up = _treesum([pr[j] * g[2 * j] for j in range(8)])
            down = _treesum([pr[j] * g[2 * j + 1] for j in range(8)])
            dec_v[pl.ds((t + MEM_LEN - 1) * LANES, LANES)] = jnp.where(
                up < down, one, zero)

        def beta_products(b, es):
            q = [b[2 * k] + b[2 * k + 1] for k in range(8)]
            return [q[s % 8] * es[s] for s in range(S)]

        def b_step8(i, b):
            nb = tuple(b)
            for k in range(7):
                j = CHUNK - 1 - 8 * i - k
                es = [_row(ebuf_v, j * S + s) for s in range(S)]
                pr = [_row(abuf_v, j * 8 + jj) for jj in range(8)]
                nb = beta_products(nb, es)
                decode(pr, es, nb, c * CHUNK + j)
            j = CHUNK - 8 - 8 * i
            es = [_row(ebuf_v, j * S + s) for s in range(S)]
            pr = [_row(abuf_v, j * 8 + jj) for jj in range(8)]
            nb = normalize(beta_products(nb, es))
            decode(pr, es, nb, c * CHUNK + j)
            return tuple(nb)

        return lax.fori_loop(0, CHUNK // 8, b_step8, tuple(b))

    lax.fori_loop(0, N_CHUNKS, bwd_chunk, onehot0)

    pltpu.sync_copy(dec_v.at[pl.ds(0, T_LEN * LANES)], out_hbm.at[wid])


@jax.jit
def _bcjr_call(yt, params):
    mesh = plsc.VectorSubcoreMesh(core_axis_name="c", subcore_axis_name="s")
    f = functools.partial(
        pl.kernel,
        mesh=mesh,
        out_type=jax.ShapeDtypeStruct((N_WORKERS, T_LEN * LANES), jnp.float32),
        scratch_types=[
            pltpu.VMEM((T_LEN * LANES,), jnp.float32),
            pltpu.VMEM(((T_LEN + MEM_LEN - 1) * LANES,), jnp.float32),
            pltpu.VMEM((N_CHUNKS * N_STATES * LANES,), jnp.float32),
            pltpu.VMEM((CHUNK * 8 * LANES,), jnp.float32),
            pltpu.VMEM((CHUNK * N_STATES * LANES,), jnp.float32),
            pltpu.VMEM(((2 * N_STATES + 2) * LANES,), jnp.float32),
        ],
    )(_bcjr_sc_kernel)
    return f(yt, params)


def kernel(y, snr, gamma):
    m = MEM_LEN
    gamma_f = jnp.asarray(gamma, jnp.float32)
    h = jnp.exp(-gamma_f * jnp.arange(m, dtype=jnp.float32))
    bits = np.unpackbits(np.arange(N_STATES, dtype=np.uint8).reshape(-1, 1),
                         axis=1)[:, -m:].astype(np.float32)
    symbols = jnp.asarray(1.0 - 2.0 * bits)
    sp = symbols @ h[::-1]
    sigma = 10.0 ** (-jnp.asarray(snr, jnp.float32) / 10.0)
    ninv2 = -1.0 / (2.0 * sigma * sigma)
    nlogc = -jnp.log(jnp.asarray(math.sqrt(2.0 * math.pi), jnp.float32) * sigma)
    c1 = -2.0 * ninv2 * sp
    c2 = ninv2 * sp * sp
    params = jnp.concatenate(
        [jnp.broadcast_to(c1[:, None], (N_STATES, LANES)),
         jnp.broadcast_to(c2[:, None], (N_STATES, LANES)),
         jnp.full((1, LANES), ninv2, jnp.float32),
         jnp.full((1, LANES), nlogc, jnp.float32)], axis=0)

    B, T = y.shape
    yt = y.reshape(N_WORKERS, LANES, T).transpose(0, 2, 1)
    raw = _bcjr_call(yt.reshape(N_WORKERS, T * LANES),
                     params.reshape((2 * N_STATES + 2) * LANES))
    return raw.reshape(N_WORKERS, T, LANES).transpose(0, 2, 1).reshape(B, T)
